# baseline (speedup 1.0000x reference)
import functools

import jax
import jax.numpy as jnp
from jax import lax
from jax.experimental import pallas as pl
from jax.experimental.pallas import tpu as pltpu
from jax.experimental.pallas import tpu_sc as plsc

_T = 16
_M = 500000
_D = 2
_B = 4096

_NC = 2
_NS = 16
_L = 16
_RANGE = _M // _NS
_POSBITS = 12
_G = _B // _L

_N = _T * _M * _D
_CBLK = 640000


def _tc_copy_body(x_ref, o_ref):
  o_ref[...] = x_ref[...]


@functools.cache
def _build_tc_copy():
  return pl.pallas_call(
      _tc_copy_body,
      out_shape=jax.ShapeDtypeStruct((_N,), jnp.float32),
      grid=(_N // _CBLK,),
      in_specs=[pl.BlockSpec((_CBLK,), lambda i: (i,))],
      out_specs=pl.BlockSpec((_CBLK,), lambda i: (i,)),
  )


def _sc_body(idx_hbm, val_hbm, out_hbm,
             lp_sh,
             idxv, aux, lpown, iotav, lpf, valrow, vbuf, offv,
             sema, sems):
  c = lax.axis_index("c")
  s = lax.axis_index("s")
  t = s
  d = c
  iota = lax.iota(jnp.int32, _L)

  pltpu.sync_copy(idx_hbm, idxv)
  pltpu.sync_copy(val_hbm.at[t, d], valrow)

  lo = s * _RANGE

  def g1(g, carry):
    a = idxv[pl.ds(g * _L, _L)]
    pos = g * _L + iota
    key = (a << _POSBITS) | pos
    ks, _ = plsc.sort_key_val(key, key)
    asort = ks >> _POSBITS
    psort = ks & (_B - 1)
    offv[pl.ds(0, _L)] = asort
    anext = plsc.load_gather(offv, [jnp.minimum(iota + 1, _L - 1)])
    rel = asort - lo
    inr = (rel >= 0) & (rel < _RANGE)
    mlast = (asort != anext) | (iota == _L - 1)
    plsc.store_scatter(aux, [jnp.where(inr, rel, 0)], psort, mask=mlast & inr)
    return carry

  lax.fori_loop(0, _G, g1, 0)

  def g2(g, carry):
    a = idxv[pl.ds(g * _L, _L)]
    rel = a - lo
    inr = (rel >= 0) & (rel < _RANGE)
    p = plsc.load_gather(aux, [jnp.where(inr, rel, 0)], mask=inr)
    lpown[pl.ds(g * _L, _L)] = jnp.where(inr, p, 0)
    lpf[pl.ds(g * _L, _L)] = iota * 0
    iotav[pl.ds(g * _L, _L)] = g * _L + iota
    return carry

  lax.fori_loop(0, _G, g2, 0)

  @pl.when(s == 0)
  def _init():
    pltpu.sync_copy(lpf, lp_sh)

  plsc.subcore_barrier()
  adds = []
  for blk in range(_B // 128):
    adds.append(pltpu.async_copy(
        lpown.at[pl.ds(blk * 128, 128)],
        lp_sh.at[iotav.at[pl.ds(blk * 128, 128)]], sema,
        add=True))
  for a_ in adds:
    a_.wait()
  plsc.subcore_barrier()
  pltpu.sync_copy(lp_sh, lpf)

  base = t * (_M * _D) + d

  def g3(g, carry):
    a = idxv[pl.ds(g * _L, _L)]
    lpv = lpf[pl.ds(g * _L, _L)]
    v = plsc.load_gather(valrow, [lpv])
    vbuf[pl.ds(g * _L, _L)] = v
    offv[pl.ds(g * _L, _L)] = base + a * _D
    return carry

  lax.fori_loop(0, _G, g3, 0)

  writes = []
  for blk in range(_B // 128):
    writes.append(pltpu.async_copy(
        vbuf.at[pl.ds(blk * 128, 128)],
        out_hbm.at[offv.at[pl.ds(blk * 128, 128)]],
        sems[blk % 8]))
  for w in writes:
    w.wait()


@functools.cache
def _build_sc_scatter():
  return pl.kernel(
      _sc_body,
      out_type=(),
      mesh=plsc.VectorSubcoreMesh(core_axis_name="c", subcore_axis_name="s",
                                  num_cores=_NC, num_subcores=_NS),
      compiler_params=pltpu.CompilerParams(needs_layout_passes=False,
                                           use_tc_tiling_on_sc=False),
      scratch_types=[
          pltpu.VMEM_SHARED((_B,), jnp.int32),
          pltpu.VMEM((_B,), jnp.int32),
          pltpu.VMEM((_RANGE,), jnp.int32),
          pltpu.VMEM((_B,), jnp.int32),
          pltpu.VMEM((_B,), jnp.int32),
          pltpu.VMEM((_B,), jnp.int32),
          pltpu.VMEM((_B,), jnp.float32),
          pltpu.VMEM((_B,), jnp.float32),
          pltpu.VMEM((_B,), jnp.int32),
          pltpu.SemaphoreType.DMA,
          [pltpu.SemaphoreType.DMA] * 8,
      ],
  )


def kernel(mem, idx, val):
  valT = val.transpose(0, 2, 1)
  flat = _build_tc_copy()(mem.reshape(_N))
  ref = jax.new_ref(flat)
  _build_sc_scatter()(idx, valT, ref)
  return ref[...].reshape(_T, _M, _D)
